# baseline (speedup 1.0000x reference)
import jax
import jax.numpy as jnp
from jax import lax
from jax.experimental import pallas as pl
from jax.experimental.pallas import tpu as pltpu
from jax.experimental.pallas import tpu_sc as plsc

_N = 10000
_NP = 10112
_NPT = 632
_D = 256
_DQ = 64
_DA = 80
_G = 64
_EDGES = 170000
_CHUNK = 64
_K2T = 5632
_EP = 32 * _K2T
_K2_ROWS = _K2T // _CHUNK
_K4C = 128
_K4T = _EP // 16
_K4_ROWS = _K4T // _K4C
_NROW = _EP // _K4C


_GATHER_DNUMS = lax.GatherDimensionNumbers(
    offset_dims=(), collapsed_slice_dims=(0,), start_index_map=(0,))


def _lane_shuffle(v, k):
    perm = jnp.bitwise_xor(lax.iota(jnp.int32, 16), k).reshape(16, 1)
    return lax.gather(v, perm, _GATHER_DNUMS, (1,),
                      mode=lax.GatherScatterMode.PROMISE_IN_BOUNDS)


def _bcast_lane(v, j):
    perm = jnp.full((16, 1), j, jnp.int32)
    return lax.gather(v, perm, _GATHER_DNUMS, (1,),
                      mode=lax.GatherScatterMode.PROMISE_IN_BOUNDS)


def _splat_sum(v):
    for k in (8, 4, 2, 1):
        v = v + _lane_shuffle(v, k)
    return v


def _splat_max(v):
    for k in (8, 4, 2, 1):
        v = jnp.maximum(v, _lane_shuffle(v, k))
    return v


def _pre_body(x_ref, wl_ref, bl_ref, wr_ref, br_ref,
              xl_ref, xr_ref, q0_ref, q1_ref, q2_ref, q3_ref):
    x = x_ref[...]
    xl = lax.dot_general(x, wl_ref[...], (((1,), (1,)), ((), ())),
                         preferred_element_type=jnp.float32) + bl_ref[...]
    xr = lax.dot_general(x, wr_ref[...], (((1,), (1,)), ((), ())),
                         preferred_element_type=jnp.float32) + br_ref[...]
    xl_ref[...] = xl
    xr_ref[...] = xr
    ones = jnp.ones((_N, _DA - _DQ), jnp.float32)
    q0_ref[...] = jnp.concatenate([xl[:, 0 * _DQ:1 * _DQ], ones], axis=1)
    q1_ref[...] = jnp.concatenate([xl[:, 1 * _DQ:2 * _DQ], ones], axis=1)
    q2_ref[...] = jnp.concatenate([xl[:, 2 * _DQ:3 * _DQ], ones], axis=1)
    q3_ref[...] = jnp.concatenate([xl[:, 3 * _DQ:4 * _DQ], ones], axis=1)


def _project(x, W_l, b_l, W_r, b_r):
    return pl.pallas_call(
        _pre_body,
        out_shape=[
            jax.ShapeDtypeStruct((_N, _D), jnp.float32),
            jax.ShapeDtypeStruct((_N, _D), jnp.float32),
        ] + [jax.ShapeDtypeStruct((_N, _DA), jnp.float32)] * 4,
    )(x, W_l, b_l.reshape(1, _D), W_r, b_r.reshape(1, _D))


def _edge_score_body(xl_hbm, xr_hbm, src_hbm, dst_hbm, att_hbm,
                     e_hbm, pmax_hbm,
                     src_v, dst_v, a0_v, a1_v, b0_v, b1_v, att_v,
                     e0_v, e1_v, m_v,
                     ga0, ga1, gb0, gb1, es0, es1):
    cid = lax.axis_index("c")
    sid = lax.axis_index("s")
    wid = sid * 2 + cid
    base = wid * _K2T
    pltpu.sync_copy(src_hbm.at[pl.ds(base, _K2T)], src_v)
    pltpu.sync_copy(dst_hbm.at[pl.ds(base, _K2T)], dst_v)
    pltpu.sync_copy(att_hbm, att_v)
    att_regs = [att_v[pl.ds(16 * u, 16)] for u in range(16)]

    a_bufs = (a0_v, a1_v)
    b_bufs = (b0_v, b1_v)
    e_bufs = (e0_v, e1_v)
    ga = (ga0, ga1)
    gb = (gb0, gb1)
    es = (es0, es1)

    def gather_pair(c, b):
        pltpu.async_copy(
            xl_hbm.at[src_v.at[pl.ds(c * _CHUNK, _CHUNK)]], a_bufs[b], ga[b])
        pltpu.async_copy(
            xr_hbm.at[dst_v.at[pl.ds(c * _CHUNK, _CHUNK)]], b_bufs[b], gb[b])

    lane = lax.iota(jnp.int32, 16)
    neg_inf = jnp.full((16,), -1e30, jnp.float32)

    def compute_chunk(g, b, m):
        base_edge = base + g * _CHUNK
        a_v, b_v = a_bufs[b], b_bufs[b]

        def group_body(k, m):
            e16 = neg_inf
            for j in range(16):
                i = k * 16 + j
                acc = jnp.zeros((16,), jnp.float32)
                for u in range(16):
                    z = a_v[i, pl.ds(16 * u, 16)] + b_v[i, pl.ds(16 * u, 16)]
                    z = jnp.maximum(z, 0.2 * z)
                    acc = acc + z * att_regs[u]
                s = _splat_sum(acc)
                e16 = jnp.where(lane == j, s, e16)
            idxv = jnp.full((16,), base_edge + k * 16, jnp.int32) + lane
            e16 = jnp.where(idxv < _EDGES, e16, neg_inf)
            e_bufs[b][pl.ds(16 * k, 16)] = e16
            return jnp.maximum(m, e16)

        return lax.fori_loop(0, _CHUNK // 16, group_body, m)

    gather_pair(0, 0)

    def pair_body(t, m):
        for b in (0, 1):
            g = 2 * t + b
            c = g + 1
            if b == 0:
                gather_pair(c, 1)
            else:
                @pl.when(c < _K2_ROWS)
                def _():
                    gather_pair(c, 0)
            pltpu.make_async_copy(
                xl_hbm.at[src_v.at[pl.ds(g * _CHUNK, _CHUNK)]],
                a_bufs[b], ga[b]).wait()
            pltpu.make_async_copy(
                xr_hbm.at[dst_v.at[pl.ds(g * _CHUNK, _CHUNK)]],
                b_bufs[b], gb[b]).wait()

            @pl.when(t > 0)
            def _():
                pltpu.make_async_copy(
                    e_bufs[b],
                    e_hbm.at[pl.ds(base + (g - 2) * _CHUNK, _CHUNK)],
                    es[b]).wait()

            m = compute_chunk(g, b, m)
            pltpu.async_copy(
                e_bufs[b], e_hbm.at[pl.ds(base + g * _CHUNK, _CHUNK)], es[b])
        return m

    m = lax.fori_loop(0, _K2_ROWS // 2, pair_body,
                      jnp.full((16,), -1e30, jnp.float32))
    for b in (0, 1):
        pltpu.make_async_copy(
            e_bufs[b],
            e_hbm.at[pl.ds(base + (_K2_ROWS - 2 + b) * _CHUNK, _CHUNK)],
            es[b]).wait()
    m_v[...] = m
    pltpu.sync_copy(m_v, pmax_hbm.at[pl.ds(wid * 16, 16)])


def _edge_scores(xl, xr, src1d, dst1d, att):
    mesh = plsc.VectorSubcoreMesh(core_axis_name="c", subcore_axis_name="s")
    fn = pl.kernel(
        _edge_score_body,
        mesh=mesh,
        out_type=[
            jax.ShapeDtypeStruct((_EP,), jnp.float32),
            jax.ShapeDtypeStruct((512,), jnp.float32),
        ],
        scratch_types=[
            pltpu.VMEM((_K2T,), jnp.int32),
            pltpu.VMEM((_K2T,), jnp.int32),
            pltpu.VMEM((_CHUNK, _D), jnp.float32),
            pltpu.VMEM((_CHUNK, _D), jnp.float32),
            pltpu.VMEM((_CHUNK, _D), jnp.float32),
            pltpu.VMEM((_CHUNK, _D), jnp.float32),
            pltpu.VMEM((_D,), jnp.float32),
            pltpu.VMEM((_CHUNK,), jnp.float32),
            pltpu.VMEM((_CHUNK,), jnp.float32),
            pltpu.VMEM((16,), jnp.float32),
        ] + [pltpu.SemaphoreType.DMA] * 6,
        compiler_params=pltpu.CompilerParams(needs_layout_passes=False),
    )
    return fn(xl, xr, src1d, dst1d, att)


def _aggregate_body(q0_hbm, q1_hbm, q2_hbm, q3_hbm, src_hbm, dst2d_hbm,
                    e_hbm, pmax_hbm,
                    out0_hbm, out1_hbm, out2_hbm, out3_hbm,
                    src_v, dst_v, e2_v,
                    r0_v, r1_v, r2_v, r3_v,
                    p_v, pm_v, zb_v, acc_sh,
                    gs0, gs1, gs2, gs3, ss0, ss1, ss2, ss3):
    cid = lax.axis_index("c")
    tid = lax.axis_index("s")
    base = tid * _K4T
    base_row = tid * _K4_ROWS
    node_base = tid * _NPT

    rows = (r0_v, r1_v, r2_v, r3_v)
    gs = (gs0, gs1, gs2, gs3)
    ss = (ss0, ss1, ss2, ss3)

    pltpu.sync_copy(pmax_hbm, pm_v)
    mv = pm_v[pl.ds(0, 16)]
    for u in range(1, 32):
        mv = jnp.maximum(mv, pm_v[pl.ds(16 * u, 16)])
    M = _splat_max(mv)

    def zrow(i, _):
        def zcol(w, _):
            zb_v[i, pl.ds(16 * w, 16)] = jnp.zeros((16,), jnp.float32)
            return 0
        lax.fori_loop(0, _DA // 16, zcol, 0)
        return 0
    lax.fori_loop(0, _CHUNK, zrow, 0)

    pltpu.sync_copy(src_hbm.at[pl.ds(base, _K4T)], src_v)
    pltpu.sync_copy(dst2d_hbm.at[pl.ds(base_row, _K4_ROWS)], dst_v)
    pltpu.sync_copy(e_hbm.at[pl.ds(base, _K4T)], e2_v)

    def run_quarter(x_q_hbm, out_hbm):
        for j in range(9):
            sl = pl.ds(node_base + j * _CHUNK, _CHUNK)
            pltpu.sync_copy(zb_v, acc_sh.at[sl])
        pltpu.sync_copy(zb_v.at[pl.ds(0, _NPT - 9 * _CHUNK)],
                        acc_sh.at[pl.ds(node_base + 9 * _CHUNK,
                                        _NPT - 9 * _CHUNK)])
        plsc.subcore_barrier()

        def fire_gather(c, b):
            pltpu.async_copy(
                x_q_hbm.at[src_v.at[pl.ds(c * _K4C, _K4C)]], rows[b], gs[b])

        def drain_scatter(c, b):
            pltpu.make_async_copy(
                rows[b], acc_sh.at[dst_v.at[c]], ss[b]).wait()

        fire_gather(0, 0)
        fire_gather(1, 1)

        def quad_body(t, _):
            for b in range(4):
                g = 4 * t + b

                @pl.when(g >= 2)
                def _():
                    drain_scatter(g - 2, (b + 2) % 4)

                @pl.when(g + 2 < _K4_ROWS)
                def _():
                    fire_gather(g + 2, (b + 2) % 4)

                pltpu.make_async_copy(
                    x_q_hbm.at[src_v.at[pl.ds(g * _K4C, _K4C)]],
                    rows[b], gs[b]).wait()

                for v in range(_K4C // 16):
                    p_v[pl.ds(16 * v, 16)] = jnp.exp(
                        e2_v[pl.ds(g * _K4C + 16 * v, 16)] - M)

                def group_body(kk, _):
                    p16 = p_v[pl.ds(16 * kk, 16)]
                    for j in range(16):
                        i = kk * 16 + j
                        pe = _bcast_lane(p16, j)
                        for u in range(_DA // 16):
                            rows[b][i, pl.ds(16 * u, 16)] = (
                                rows[b][i, pl.ds(16 * u, 16)] * pe)
                    return 0

                lax.fori_loop(0, _K4C // 16, group_body, 0)
                pltpu.async_copy(rows[b], acc_sh.at[dst_v.at[g]], ss[b],
                                 add=True)
            return 0

        lax.fori_loop(0, _K4_ROWS // 4, quad_body, 0)
        drain_scatter(_K4_ROWS - 2, 2)
        drain_scatter(_K4_ROWS - 1, 3)
        plsc.subcore_barrier()
        for j in range(9):
            sl = pl.ds(node_base + j * _CHUNK, _CHUNK)
            pltpu.sync_copy(acc_sh.at[sl], out_hbm.at[sl])
        sl = pl.ds(node_base + 9 * _CHUNK, _NPT - 9 * _CHUNK)
        pltpu.sync_copy(acc_sh.at[sl], out_hbm.at[sl])

    @pl.when(cid == 0)
    def _():
        run_quarter(q0_hbm, out0_hbm)
        run_quarter(q1_hbm, out1_hbm)

    @pl.when(cid == 1)
    def _():
        run_quarter(q2_hbm, out2_hbm)
        run_quarter(q3_hbm, out3_hbm)


def _aggregate(xq, src1d, dst2d, e1d, pmax):
    mesh = plsc.VectorSubcoreMesh(core_axis_name="c", subcore_axis_name="s")
    fn = pl.kernel(
        _aggregate_body,
        mesh=mesh,
        out_type=[jax.ShapeDtypeStruct((_NP, _DA), jnp.float32)] * 4,
        scratch_types=[
            pltpu.VMEM((_K4T,), jnp.int32),
            pltpu.VMEM((_K4_ROWS, _K4C), jnp.int32),
            pltpu.VMEM((_K4T,), jnp.float32),
        ] + [pltpu.VMEM((_K4C, _DA), jnp.float32)] * 4 + [
            pltpu.VMEM((_K4C,), jnp.float32),
            pltpu.VMEM((512,), jnp.float32),
            pltpu.VMEM((_CHUNK, _DA), jnp.float32),
            pltpu.VMEM_SHARED((_NP, _DA), jnp.float32),
        ] + [pltpu.SemaphoreType.DMA] * 8,
        compiler_params=pltpu.CompilerParams(needs_layout_passes=False,
                                             use_tc_tiling_on_sc=False),
    )
    return fn(xq[0], xq[1], xq[2], xq[3], src1d, dst2d, e1d, pmax)


def _post_body(q0_ref, q1_ref, q2_ref, q3_ref, batch_ref, bias_ref,
               wlin_ref, blin_ref, out_ref):
    den = q0_ref[...][:_N, _DQ:_DQ + 1] + 1e-16
    h = jnp.concatenate(
        [q0_ref[...][:_N, :_DQ], q1_ref[...][:_N, :_DQ],
         q2_ref[...][:_N, :_DQ], q3_ref[...][:_N, :_DQ]], axis=1) / den
    h = h + bias_ref[...]
    oh = (batch_ref[...] ==
          lax.broadcasted_iota(jnp.int32, (_N, _G), 1)).astype(jnp.float32)
    sums = lax.dot_general(oh, h, (((0,), (0,)), ((), ())),
                           preferred_element_type=jnp.float32)
    counts = lax.dot_general(oh, jnp.ones((_N, 1), jnp.float32),
                             (((0,), (0,)), ((), ())),
                             preferred_element_type=jnp.float32)
    pooled = sums / jnp.maximum(counts, 1.0)
    out_ref[...] = lax.dot_general(pooled, wlin_ref[...],
                                   (((1,), (1,)), ((), ())),
                                   preferred_element_type=jnp.float32) + blin_ref[...]


def _finish(accs, batch, bias_conv, W_lin, b_lin):
    return pl.pallas_call(
        _post_body,
        out_shape=jax.ShapeDtypeStruct((_G, _D), jnp.float32),
    )(accs[0], accs[1], accs[2], accs[3], batch.reshape(_N, 1),
      bias_conv.reshape(1, _D), W_lin, b_lin.reshape(1, _D))


def kernel(x, edge_index, edge_attr, batch, W_l, b_l, W_r, b_r, att,
           bias_conv, W_lin, b_lin):
    del edge_attr
    loop = jnp.arange(_N, dtype=jnp.int32)
    pad = jnp.zeros((_EP - _EDGES,), jnp.int32)
    src1d = jnp.concatenate([edge_index[0], loop, pad])
    dst1d = jnp.concatenate([edge_index[1], loop, pad])
    dst2d = dst1d.reshape(_NROW, _K4C)

    xl, xr, q0, q1, q2, q3 = _project(x, W_l, b_l, W_r, b_r)
    e1d, pmax = _edge_scores(xl, xr, src1d, dst1d, att)
    a0, a1, a2, a3 = _aggregate((q0, q1, q2, q3), src1d, dst2d, e1d, pmax)
    return _finish((a0, a1, a2, a3), batch, bias_conv, W_lin, b_lin)


# speedup vs baseline: 2.6905x; 2.6905x over previous
import jax
import jax.numpy as jnp
from jax import lax
from jax.experimental import pallas as pl
from jax.experimental.pallas import tpu as pltpu
from jax.experimental.pallas import tpu_sc as plsc

_N = 10000
_NP = 10112
_NPT = 632
_D = 256
_DQ = 64
_DA = 80
_G = 64
_EDGES = 170000
_CHUNK = 64
_K2T = 5376
_EP = 32 * _K2T
_K2_ROWS = _K2T // _CHUNK
_K4C = 128
_K4T = _EP // 16
_K4_ROWS = _K4T // _K4C
_NROW = _EP // _K4C


_GATHER_DNUMS = lax.GatherDimensionNumbers(
    offset_dims=(), collapsed_slice_dims=(0,), start_index_map=(0,))


def _lane_shuffle(v, k):
    perm = jnp.bitwise_xor(lax.iota(jnp.int32, 16), k).reshape(16, 1)
    return lax.gather(v, perm, _GATHER_DNUMS, (1,),
                      mode=lax.GatherScatterMode.PROMISE_IN_BOUNDS)


def _bcast_lane(v, j):
    perm = jnp.full((16, 1), j, jnp.int32)
    return lax.gather(v, perm, _GATHER_DNUMS, (1,),
                      mode=lax.GatherScatterMode.PROMISE_IN_BOUNDS)


def _splat_sum(v):
    for k in (8, 4, 2, 1):
        v = v + _lane_shuffle(v, k)
    return v


def _splat_max(v):
    for k in (8, 4, 2, 1):
        v = jnp.maximum(v, _lane_shuffle(v, k))
    return v


def _pre_body(x_ref, wl_ref, bl_ref, wr_ref, br_ref,
              xl_ref, xr_ref, q0_ref, q1_ref, q2_ref, q3_ref):
    x = x_ref[...]
    xl = lax.dot_general(x, wl_ref[...], (((1,), (1,)), ((), ())),
                         preferred_element_type=jnp.float32) + bl_ref[...]
    xr = lax.dot_general(x, wr_ref[...], (((1,), (1,)), ((), ())),
                         preferred_element_type=jnp.float32) + br_ref[...]
    xl_ref[...] = xl.astype(jnp.bfloat16)
    xr_ref[...] = xr.astype(jnp.bfloat16)
    ones = jnp.ones((_N, _DA - _DQ), jnp.float32)
    q0_ref[...] = jnp.concatenate([xl[:, 0 * _DQ:1 * _DQ], ones], axis=1)
    q1_ref[...] = jnp.concatenate([xl[:, 1 * _DQ:2 * _DQ], ones], axis=1)
    q2_ref[...] = jnp.concatenate([xl[:, 2 * _DQ:3 * _DQ], ones], axis=1)
    q3_ref[...] = jnp.concatenate([xl[:, 3 * _DQ:4 * _DQ], ones], axis=1)


def _project(x, W_l, b_l, W_r, b_r):
    return pl.pallas_call(
        _pre_body,
        out_shape=[
            jax.ShapeDtypeStruct((_N, _D), jnp.bfloat16),
            jax.ShapeDtypeStruct((_N, _D), jnp.bfloat16),
        ] + [jax.ShapeDtypeStruct((_N, _DA), jnp.float32)] * 4,
    )(x, W_l, b_l.reshape(1, _D), W_r, b_r.reshape(1, _D))


def _edge_score_body(xl_hbm, xr_hbm, src_hbm, dst_hbm, att_hbm,
                     e_hbm, pmax_hbm,
                     src_v, dst_v, a0_v, a1_v, b0_v, b1_v, att_v,
                     e0_v, e1_v, m_v,
                     ga0, ga1, gb0, gb1, es0, es1):
    cid = lax.axis_index("c")
    sid = lax.axis_index("s")
    wid = sid * 2 + cid
    base = wid * _K2T
    pltpu.sync_copy(src_hbm.at[pl.ds(base, _K2T)], src_v)
    pltpu.sync_copy(dst_hbm.at[pl.ds(base, _K2T)], dst_v)
    pltpu.sync_copy(att_hbm, att_v)
    att_regs = [
        plsc.unpack(att_v[pl.ds(32 * u, 32)],
                    format=plsc.PackFormat.INTERLEAVED,
                    preferred_element_type=jnp.float32)
        for u in range(8)
    ]

    a_bufs = (a0_v, a1_v)
    b_bufs = (b0_v, b1_v)
    e_bufs = (e0_v, e1_v)
    ga = (ga0, ga1)
    gb = (gb0, gb1)
    es = (es0, es1)

    def gather_pair(c, b):
        pltpu.async_copy(
            xl_hbm.at[src_v.at[pl.ds(c * _CHUNK, _CHUNK)]], a_bufs[b], ga[b])
        pltpu.async_copy(
            xr_hbm.at[dst_v.at[pl.ds(c * _CHUNK, _CHUNK)]], b_bufs[b], gb[b])

    lane = lax.iota(jnp.int32, 16)
    neg_inf = jnp.full((16,), -1e30, jnp.float32)

    def compute_chunk(g, b, m):
        base_edge = base + g * _CHUNK
        a_v, b_v = a_bufs[b], b_bufs[b]

        def group_body(k, m):
            e16 = neg_inf
            for j in range(16):
                i = k * 16 + j
                acc = jnp.zeros((16,), jnp.float32)
                for u in range(8):
                    z = a_v[i, pl.ds(32 * u, 32)] + b_v[i, pl.ds(32 * u, 32)]
                    z0, z1 = plsc.unpack(
                        z, format=plsc.PackFormat.INTERLEAVED,
                        preferred_element_type=jnp.float32)
                    z0 = jnp.maximum(z0, 0.2 * z0)
                    z1 = jnp.maximum(z1, 0.2 * z1)
                    acc = acc + z0 * att_regs[u][0] + z1 * att_regs[u][1]
                s = _splat_sum(acc)
                e16 = jnp.where(lane == j, s, e16)
            idxv = jnp.full((16,), base_edge + k * 16, jnp.int32) + lane
            e16 = jnp.where(idxv < _EDGES, e16, neg_inf)
            e_bufs[b][pl.ds(16 * k, 16)] = e16
            return jnp.maximum(m, e16)

        return lax.fori_loop(0, _CHUNK // 16, group_body, m)

    gather_pair(0, 0)

    def pair_body(t, m):
        for b in (0, 1):
            g = 2 * t + b
            c = g + 1
            if b == 0:
                gather_pair(c, 1)
            else:
                @pl.when(c < _K2_ROWS)
                def _():
                    gather_pair(c, 0)
            pltpu.make_async_copy(
                xl_hbm.at[src_v.at[pl.ds(g * _CHUNK, _CHUNK)]],
                a_bufs[b], ga[b]).wait()
            pltpu.make_async_copy(
                xr_hbm.at[dst_v.at[pl.ds(g * _CHUNK, _CHUNK)]],
                b_bufs[b], gb[b]).wait()

            @pl.when(t > 0)
            def _():
                pltpu.make_async_copy(
                    e_bufs[b],
                    e_hbm.at[pl.ds(base + (g - 2) * _CHUNK, _CHUNK)],
                    es[b]).wait()

            m = compute_chunk(g, b, m)
            pltpu.async_copy(
                e_bufs[b], e_hbm.at[pl.ds(base + g * _CHUNK, _CHUNK)], es[b])
        return m

    m = lax.fori_loop(0, _K2_ROWS // 2, pair_body,
                      jnp.full((16,), -1e30, jnp.float32))
    for b in (0, 1):
        pltpu.make_async_copy(
            e_bufs[b],
            e_hbm.at[pl.ds(base + (_K2_ROWS - 2 + b) * _CHUNK, _CHUNK)],
            es[b]).wait()
    m_v[...] = m
    pltpu.sync_copy(m_v, pmax_hbm.at[pl.ds(wid * 16, 16)])


def _edge_scores(xl, xr, src1d, dst1d, att):
    mesh = plsc.VectorSubcoreMesh(core_axis_name="c", subcore_axis_name="s")
    fn = pl.kernel(
        _edge_score_body,
        mesh=mesh,
        out_type=[
            jax.ShapeDtypeStruct((_EP,), jnp.float32),
            jax.ShapeDtypeStruct((512,), jnp.float32),
        ],
        scratch_types=[
            pltpu.VMEM((_K2T,), jnp.int32),
            pltpu.VMEM((_K2T,), jnp.int32),
            pltpu.VMEM((_CHUNK, _D), jnp.bfloat16),
            pltpu.VMEM((_CHUNK, _D), jnp.bfloat16),
            pltpu.VMEM((_CHUNK, _D), jnp.bfloat16),
            pltpu.VMEM((_CHUNK, _D), jnp.bfloat16),
            pltpu.VMEM((_D,), jnp.bfloat16),
            pltpu.VMEM((_CHUNK,), jnp.float32),
            pltpu.VMEM((_CHUNK,), jnp.float32),
            pltpu.VMEM((16,), jnp.float32),
        ] + [pltpu.SemaphoreType.DMA] * 6,
        compiler_params=pltpu.CompilerParams(needs_layout_passes=False,
                                             use_tc_tiling_on_sc=False),
    )
    return fn(xl, xr, src1d, dst1d, att)


def _aggregate_body(q0_hbm, q1_hbm, q2_hbm, q3_hbm, src_hbm, dst2d_hbm,
                    e_hbm, pmax_hbm,
                    out0_hbm, out1_hbm, out2_hbm, out3_hbm,
                    src_v, dst_v, e2_v,
                    r0_v, r1_v, r2_v, r3_v,
                    p_v, pm_v, zb_v, acc_sh,
                    gs0, gs1, gs2, gs3, ss0, ss1, ss2, ss3):
    cid = lax.axis_index("c")
    tid = lax.axis_index("s")
    base = tid * _K4T
    base_row = tid * _K4_ROWS
    node_base = tid * _NPT

    rows = (r0_v, r1_v, r2_v, r3_v)
    gs = (gs0, gs1, gs2, gs3)
    ss = (ss0, ss1, ss2, ss3)

    pltpu.sync_copy(pmax_hbm, pm_v)
    mv = pm_v[pl.ds(0, 16)]
    for u in range(1, 32):
        mv = jnp.maximum(mv, pm_v[pl.ds(16 * u, 16)])
    M = _splat_max(mv)

    def zrow(i, _):
        def zcol(w, _):
            zb_v[i, pl.ds(16 * w, 16)] = jnp.zeros((16,), jnp.float32)
            return 0
        lax.fori_loop(0, _DA // 16, zcol, 0)
        return 0
    lax.fori_loop(0, _CHUNK, zrow, 0)

    pltpu.sync_copy(src_hbm.at[pl.ds(base, _K4T)], src_v)
    pltpu.sync_copy(dst2d_hbm.at[pl.ds(base_row, _K4_ROWS)], dst_v)
    pltpu.sync_copy(e_hbm.at[pl.ds(base, _K4T)], e2_v)

    def run_quarter(x_q_hbm, out_hbm):
        for j in range(9):
            sl = pl.ds(node_base + j * _CHUNK, _CHUNK)
            pltpu.sync_copy(zb_v, acc_sh.at[sl])
        pltpu.sync_copy(zb_v.at[pl.ds(0, _NPT - 9 * _CHUNK)],
                        acc_sh.at[pl.ds(node_base + 9 * _CHUNK,
                                        _NPT - 9 * _CHUNK)])
        plsc.subcore_barrier()

        def fire_gather(c, b):
            pltpu.async_copy(
                x_q_hbm.at[src_v.at[pl.ds(c * _K4C, _K4C)]], rows[b], gs[b])

        def drain_scatter(c, b):
            pltpu.make_async_copy(
                rows[b], acc_sh.at[dst_v.at[c]], ss[b]).wait()

        fire_gather(0, 0)
        fire_gather(1, 1)

        def quad_body(t, _):
            for b in range(4):
                g = 4 * t + b

                @pl.when(g >= 2)
                def _():
                    drain_scatter(g - 2, (b + 2) % 4)

                @pl.when(g + 2 < _K4_ROWS)
                def _():
                    fire_gather(g + 2, (b + 2) % 4)

                pltpu.make_async_copy(
                    x_q_hbm.at[src_v.at[pl.ds(g * _K4C, _K4C)]],
                    rows[b], gs[b]).wait()

                for v in range(_K4C // 16):
                    p_v[pl.ds(16 * v, 16)] = jnp.exp(
                        e2_v[pl.ds(g * _K4C + 16 * v, 16)] - M)

                def group_body(kk, _):
                    p16 = p_v[pl.ds(16 * kk, 16)]
                    for j in range(16):
                        i = kk * 16 + j
                        pe = _bcast_lane(p16, j)
                        for u in range(_DA // 16):
                            rows[b][i, pl.ds(16 * u, 16)] = (
                                rows[b][i, pl.ds(16 * u, 16)] * pe)
                    return 0

                lax.fori_loop(0, _K4C // 16, group_body, 0)
                pltpu.async_copy(rows[b], acc_sh.at[dst_v.at[g]], ss[b],
                                 add=True)
            return 0

        lax.fori_loop(0, _K4_ROWS // 4, quad_body, 0)
        drain_scatter(_K4_ROWS - 2, 2)
        drain_scatter(_K4_ROWS - 1, 3)
        plsc.subcore_barrier()
        for j in range(9):
            sl = pl.ds(node_base + j * _CHUNK, _CHUNK)
            pltpu.sync_copy(acc_sh.at[sl], out_hbm.at[sl])
        sl = pl.ds(node_base + 9 * _CHUNK, _NPT - 9 * _CHUNK)
        pltpu.sync_copy(acc_sh.at[sl], out_hbm.at[sl])

    @pl.when(cid == 0)
    def _():
        run_quarter(q0_hbm, out0_hbm)
        run_quarter(q1_hbm, out1_hbm)

    @pl.when(cid == 1)
    def _():
        run_quarter(q2_hbm, out2_hbm)
        run_quarter(q3_hbm, out3_hbm)


def _aggregate(xq, src1d, dst2d, e1d, pmax):
    mesh = plsc.VectorSubcoreMesh(core_axis_name="c", subcore_axis_name="s")
    fn = pl.kernel(
        _aggregate_body,
        mesh=mesh,
        out_type=[jax.ShapeDtypeStruct((_NP, _DA), jnp.float32)] * 4,
        scratch_types=[
            pltpu.VMEM((_K4T,), jnp.int32),
            pltpu.VMEM((_K4_ROWS, _K4C), jnp.int32),
            pltpu.VMEM((_K4T,), jnp.float32),
        ] + [pltpu.VMEM((_K4C, _DA), jnp.float32)] * 4 + [
            pltpu.VMEM((_K4C,), jnp.float32),
            pltpu.VMEM((512,), jnp.float32),
            pltpu.VMEM((_CHUNK, _DA), jnp.float32),
            pltpu.VMEM_SHARED((_NP, _DA), jnp.float32),
        ] + [pltpu.SemaphoreType.DMA] * 8,
        compiler_params=pltpu.CompilerParams(needs_layout_passes=False,
                                             use_tc_tiling_on_sc=False),
    )
    return fn(xq[0], xq[1], xq[2], xq[3], src1d, dst2d, e1d, pmax)


def _post_body(q0_ref, q1_ref, q2_ref, q3_ref, batch_ref, bias_ref,
               wlin_ref, blin_ref, out_ref):
    den = q0_ref[...][:_N, _DQ:_DQ + 1] + 1e-16
    h = jnp.concatenate(
        [q0_ref[...][:_N, :_DQ], q1_ref[...][:_N, :_DQ],
         q2_ref[...][:_N, :_DQ], q3_ref[...][:_N, :_DQ]], axis=1) / den
    h = h + bias_ref[...]
    oh = (batch_ref[...] ==
          lax.broadcasted_iota(jnp.int32, (_N, _G), 1)).astype(jnp.float32)
    sums = lax.dot_general(oh, h, (((0,), (0,)), ((), ())),
                           preferred_element_type=jnp.float32)
    counts = lax.dot_general(oh, jnp.ones((_N, 1), jnp.float32),
                             (((0,), (0,)), ((), ())),
                             preferred_element_type=jnp.float32)
    pooled = sums / jnp.maximum(counts, 1.0)
    out_ref[...] = lax.dot_general(pooled, wlin_ref[...],
                                   (((1,), (1,)), ((), ())),
                                   preferred_element_type=jnp.float32) + blin_ref[...]


def _finish(accs, batch, bias_conv, W_lin, b_lin):
    return pl.pallas_call(
        _post_body,
        out_shape=jax.ShapeDtypeStruct((_G, _D), jnp.float32),
    )(accs[0], accs[1], accs[2], accs[3], batch.reshape(_N, 1),
      bias_conv.reshape(1, _D), W_lin, b_lin.reshape(1, _D))


def kernel(x, edge_index, edge_attr, batch, W_l, b_l, W_r, b_r, att,
           bias_conv, W_lin, b_lin):
    del edge_attr
    loop = jnp.arange(_N, dtype=jnp.int32)
    pad = jnp.zeros((_EP - _EDGES,), jnp.int32)
    src1d = jnp.concatenate([edge_index[0], loop, pad])
    dst1d = jnp.concatenate([edge_index[1], loop, pad])
    dst2d = dst1d.reshape(_NROW, _K4C)

    xl, xr, q0, q1, q2, q3 = _project(x, W_l, b_l, W_r, b_r)
    e1d, pmax = _edge_scores(xl, xr, src1d, dst1d,
                             att.astype(jnp.bfloat16))
    a0, a1, a2, a3 = _aggregate((q0, q1, q2, q3), src1d, dst2d, e1d, pmax)
    return _finish((a0, a1, a2, a3), batch, bias_conv, W_lin, b_lin)


# speedup vs baseline: 3.8053x; 1.4143x over previous
import jax
import jax.numpy as jnp
from jax import lax
from jax.experimental import pallas as pl
from jax.experimental.pallas import tpu as pltpu
from jax.experimental.pallas import tpu_sc as plsc

_N = 10000
_NP = 10112
_NPT = 632
_D = 256
_DQ = 64
_DA = 80
_G = 64
_EDGES = 170000
_CHUNK = 64
_K2T = 5376
_EP = 32 * _K2T
_K2_ROWS = _K2T // _CHUNK
_K4C = 128
_K4T = _EP // 16
_K4_ROWS = _K4T // _K4C
_NROW = _EP // _K4C


_GATHER_DNUMS = lax.GatherDimensionNumbers(
    offset_dims=(), collapsed_slice_dims=(0,), start_index_map=(0,))


def _lane_shuffle(v, k):
    perm = jnp.bitwise_xor(lax.iota(jnp.int32, 16), k).reshape(16, 1)
    return lax.gather(v, perm, _GATHER_DNUMS, (1,),
                      mode=lax.GatherScatterMode.PROMISE_IN_BOUNDS)


def _bcast_lane(v, j):
    perm = jnp.full((16, 1), j, jnp.int32)
    return lax.gather(v, perm, _GATHER_DNUMS, (1,),
                      mode=lax.GatherScatterMode.PROMISE_IN_BOUNDS)


def _splat_sum(v):
    for k in (8, 4, 2, 1):
        v = v + _lane_shuffle(v, k)
    return v


def _splat_max(v):
    for k in (8, 4, 2, 1):
        v = jnp.maximum(v, _lane_shuffle(v, k))
    return v


def _pre_body(x_ref, wl_ref, bl_ref, wr_ref, br_ref,
              xl_ref, xr_ref, q0_ref, q1_ref, q2_ref, q3_ref):
    x = x_ref[...]
    xl = lax.dot_general(x, wl_ref[...], (((1,), (1,)), ((), ())),
                         preferred_element_type=jnp.float32) + bl_ref[...]
    xr = lax.dot_general(x, wr_ref[...], (((1,), (1,)), ((), ())),
                         preferred_element_type=jnp.float32) + br_ref[...]
    xl_ref[...] = xl.astype(jnp.bfloat16)
    xr_ref[...] = xr.astype(jnp.bfloat16)
    ones = jnp.ones((_N, _DA - _DQ), jnp.float32)
    q0_ref[...] = jnp.concatenate([xl[:, 0 * _DQ:1 * _DQ], ones], axis=1)
    q1_ref[...] = jnp.concatenate([xl[:, 1 * _DQ:2 * _DQ], ones], axis=1)
    q2_ref[...] = jnp.concatenate([xl[:, 2 * _DQ:3 * _DQ], ones], axis=1)
    q3_ref[...] = jnp.concatenate([xl[:, 3 * _DQ:4 * _DQ], ones], axis=1)


def _project(x, W_l, b_l, W_r, b_r):
    return pl.pallas_call(
        _pre_body,
        out_shape=[
            jax.ShapeDtypeStruct((_N, _D), jnp.bfloat16),
            jax.ShapeDtypeStruct((_N, _D), jnp.bfloat16),
        ] + [jax.ShapeDtypeStruct((_N, _DA), jnp.float32)] * 4,
    )(x, W_l, b_l.reshape(1, _D), W_r, b_r.reshape(1, _D))


def _edge_score_body(xl_hbm, xr_hbm, src_hbm, dst_hbm, att_hbm,
                     e_hbm, pmax_hbm,
                     src_v, dst_v, a0_v, a1_v, b0_v, b1_v, att_v,
                     e0_v, e1_v, m_v,
                     ga0, ga1, gb0, gb1, es0, es1):
    cid = lax.axis_index("c")
    sid = lax.axis_index("s")
    wid = sid * 2 + cid
    base = wid * _K2T
    pltpu.sync_copy(src_hbm.at[pl.ds(base, _K2T)], src_v)
    pltpu.sync_copy(dst_hbm.at[pl.ds(base, _K2T)], dst_v)
    pltpu.sync_copy(att_hbm, att_v)
    att_regs = [
        plsc.unpack(att_v[pl.ds(32 * u, 32)],
                    format=plsc.PackFormat.INTERLEAVED,
                    preferred_element_type=jnp.float32)
        for u in range(8)
    ]

    a_bufs = (a0_v, a1_v)
    b_bufs = (b0_v, b1_v)
    e_bufs = (e0_v, e1_v)
    ga = (ga0, ga1)
    gb = (gb0, gb1)
    es = (es0, es1)

    def gather_pair(c, b):
        pltpu.async_copy(
            xl_hbm.at[src_v.at[pl.ds(c * _CHUNK, _CHUNK)]], a_bufs[b], ga[b])
        pltpu.async_copy(
            xr_hbm.at[dst_v.at[pl.ds(c * _CHUNK, _CHUNK)]], b_bufs[b], gb[b])

    lane = lax.iota(jnp.int32, 16)
    neg_inf = jnp.full((16,), -1e30, jnp.float32)

    def compute_chunk(g, b, m):
        base_edge = base + g * _CHUNK
        a_v, b_v = a_bufs[b], b_bufs[b]

        def group_body(k, m):
            e16 = neg_inf
            for j in range(16):
                i = k * 16 + j
                acc = jnp.zeros((16,), jnp.float32)
                for u in range(8):
                    z = a_v[i, pl.ds(32 * u, 32)] + b_v[i, pl.ds(32 * u, 32)]
                    z0, z1 = plsc.unpack(
                        z, format=plsc.PackFormat.INTERLEAVED,
                        preferred_element_type=jnp.float32)
                    z0 = jnp.maximum(z0, 0.2 * z0)
                    z1 = jnp.maximum(z1, 0.2 * z1)
                    acc = acc + z0 * att_regs[u][0] + z1 * att_regs[u][1]
                s = _splat_sum(acc)
                e16 = jnp.where(lane == j, s, e16)
            idxv = jnp.full((16,), base_edge + k * 16, jnp.int32) + lane
            e16 = jnp.where(idxv < _EDGES, e16, neg_inf)
            e_bufs[b][pl.ds(16 * k, 16)] = e16
            return jnp.maximum(m, e16)

        return lax.fori_loop(0, _CHUNK // 16, group_body, m)

    gather_pair(0, 0)

    def pair_body(t, m):
        for b in (0, 1):
            g = 2 * t + b
            c = g + 1
            if b == 0:
                gather_pair(c, 1)
            else:
                @pl.when(c < _K2_ROWS)
                def _():
                    gather_pair(c, 0)
            pltpu.make_async_copy(
                xl_hbm.at[src_v.at[pl.ds(g * _CHUNK, _CHUNK)]],
                a_bufs[b], ga[b]).wait()
            pltpu.make_async_copy(
                xr_hbm.at[dst_v.at[pl.ds(g * _CHUNK, _CHUNK)]],
                b_bufs[b], gb[b]).wait()

            @pl.when(t > 0)
            def _():
                pltpu.make_async_copy(
                    e_bufs[b],
                    e_hbm.at[pl.ds(base + (g - 2) * _CHUNK, _CHUNK)],
                    es[b]).wait()

            m = compute_chunk(g, b, m)
            pltpu.async_copy(
                e_bufs[b], e_hbm.at[pl.ds(base + g * _CHUNK, _CHUNK)], es[b])
        return m

    m = lax.fori_loop(0, _K2_ROWS // 2, pair_body,
                      jnp.full((16,), -1e30, jnp.float32))
    for b in (0, 1):
        pltpu.make_async_copy(
            e_bufs[b],
            e_hbm.at[pl.ds(base + (_K2_ROWS - 2 + b) * _CHUNK, _CHUNK)],
            es[b]).wait()
    m_v[...] = m
    pltpu.sync_copy(m_v, pmax_hbm.at[pl.ds(wid * 16, 16)])


def _edge_scores(xl, xr, src1d, dst1d, att):
    mesh = plsc.VectorSubcoreMesh(core_axis_name="c", subcore_axis_name="s")
    fn = pl.kernel(
        _edge_score_body,
        mesh=mesh,
        out_type=[
            jax.ShapeDtypeStruct((_EP,), jnp.float32),
            jax.ShapeDtypeStruct((512,), jnp.float32),
        ],
        scratch_types=[
            pltpu.VMEM((_K2T,), jnp.int32),
            pltpu.VMEM((_K2T,), jnp.int32),
            pltpu.VMEM((_CHUNK, _D), jnp.bfloat16),
            pltpu.VMEM((_CHUNK, _D), jnp.bfloat16),
            pltpu.VMEM((_CHUNK, _D), jnp.bfloat16),
            pltpu.VMEM((_CHUNK, _D), jnp.bfloat16),
            pltpu.VMEM((_D,), jnp.bfloat16),
            pltpu.VMEM((_CHUNK,), jnp.float32),
            pltpu.VMEM((_CHUNK,), jnp.float32),
            pltpu.VMEM((16,), jnp.float32),
        ] + [pltpu.SemaphoreType.DMA] * 6,
        compiler_params=pltpu.CompilerParams(needs_layout_passes=False,
                                             use_tc_tiling_on_sc=False),
    )
    return fn(xl, xr, src1d, dst1d, att)


def _aggregate_body(q0_hbm, q1_hbm, q2_hbm, q3_hbm, src_hbm, dst2d_hbm,
                    e_hbm, pmax_hbm,
                    out0_hbm, out1_hbm, out2_hbm, out3_hbm,
                    src_v, dst_v, e2_v,
                    r0_v, r1_v, r2_v, r3_v,
                    p_v, pm_v, zb_v, acc_sh,
                    gs0, gs1, gs2, gs3, ss0, ss1, ss2, ss3):
    cid = lax.axis_index("c")
    tid = lax.axis_index("s")
    base = tid * _K4T
    base_row = tid * _K4_ROWS
    node_base = tid * _NPT

    rows = (r0_v, r1_v, r2_v, r3_v)
    gs = (gs0, gs1, gs2, gs3)
    ss = (ss0, ss1, ss2, ss3)

    pltpu.sync_copy(pmax_hbm, pm_v)
    mv = pm_v[pl.ds(0, 16)]
    for u in range(1, 32):
        mv = jnp.maximum(mv, pm_v[pl.ds(16 * u, 16)])
    M = _splat_max(mv)

    def zrow(i, _):
        def zcol(w, _):
            zb_v[i, pl.ds(16 * w, 16)] = jnp.zeros((16,), jnp.float32)
            return 0
        lax.fori_loop(0, _DA // 16, zcol, 0)
        return 0
    lax.fori_loop(0, _CHUNK, zrow, 0)

    pltpu.sync_copy(src_hbm.at[pl.ds(base, _K4T)], src_v)
    pltpu.sync_copy(dst2d_hbm.at[pl.ds(base_row, _K4_ROWS)], dst_v)
    pltpu.sync_copy(e_hbm.at[pl.ds(base, _K4T)], e2_v)

    def run_quarter(x_q_hbm, out_hbm):
        for j in range(9):
            sl = pl.ds(node_base + j * _CHUNK, _CHUNK)
            pltpu.sync_copy(zb_v, acc_sh.at[sl])
        pltpu.sync_copy(zb_v.at[pl.ds(0, _NPT - 9 * _CHUNK)],
                        acc_sh.at[pl.ds(node_base + 9 * _CHUNK,
                                        _NPT - 9 * _CHUNK)])
        plsc.subcore_barrier()

        def fire_gather(c, b):
            pltpu.async_copy(
                x_q_hbm.at[src_v.at[pl.ds(c * _K4C, _K4C)]], rows[b], gs[b])

        def drain_scatter(c, b):
            pltpu.make_async_copy(
                rows[b], acc_sh.at[dst_v.at[c]], ss[b]).wait()

        fire_gather(0, 0)
        fire_gather(1, 1)

        def quad_body(t, _):
            for b in range(4):
                g = 4 * t + b

                @pl.when(g >= 2)
                def _():
                    drain_scatter(g - 2, (b + 2) % 4)

                @pl.when(g + 2 < _K4_ROWS)
                def _():
                    fire_gather(g + 2, (b + 2) % 4)

                pltpu.make_async_copy(
                    x_q_hbm.at[src_v.at[pl.ds(g * _K4C, _K4C)]],
                    rows[b], gs[b]).wait()

                for v in range(_K4C // 16):
                    p_v[pl.ds(16 * v, 16)] = jnp.exp(
                        e2_v[pl.ds(g * _K4C + 16 * v, 16)] - M)

                def group_body(kk, _):
                    p16 = p_v[pl.ds(16 * kk, 16)]
                    for j in range(16):
                        i = kk * 16 + j
                        pe = _bcast_lane(p16, j)
                        for u in range(_DA // 16):
                            rows[b][i, pl.ds(16 * u, 16)] = (
                                rows[b][i, pl.ds(16 * u, 16)] * pe)
                    return 0

                lax.fori_loop(0, _K4C // 16, group_body, 0)
                pltpu.async_copy(rows[b], acc_sh.at[dst_v.at[g]], ss[b],
                                 add=True)
            return 0

        lax.fori_loop(0, _K4_ROWS // 4, quad_body, 0)
        drain_scatter(_K4_ROWS - 2, 2)
        drain_scatter(_K4_ROWS - 1, 3)
        plsc.subcore_barrier()
        for j in range(9):
            sl = pl.ds(node_base + j * _CHUNK, _CHUNK)
            pltpu.sync_copy(acc_sh.at[sl], out_hbm.at[sl])
        sl = pl.ds(node_base + 9 * _CHUNK, _NPT - 9 * _CHUNK)
        pltpu.sync_copy(acc_sh.at[sl], out_hbm.at[sl])

    @pl.when(cid == 0)
    def _():
        run_quarter(q0_hbm, out0_hbm)
        run_quarter(q1_hbm, out1_hbm)

    @pl.when(cid == 1)
    def _():
        run_quarter(q2_hbm, out2_hbm)
        run_quarter(q3_hbm, out3_hbm)


def _aggregate(xq, src1d, dst2d, e1d, pmax):
    mesh = plsc.VectorSubcoreMesh(core_axis_name="c", subcore_axis_name="s")
    fn = pl.kernel(
        _aggregate_body,
        mesh=mesh,
        out_type=[jax.ShapeDtypeStruct((_NP, _DA), jnp.float32)] * 4,
        scratch_types=[
            pltpu.VMEM((_K4T,), jnp.int32),
            pltpu.VMEM((_K4_ROWS, _K4C), jnp.int32),
            pltpu.VMEM((_K4T,), jnp.float32),
        ] + [pltpu.VMEM((_K4C, _DA), jnp.float32)] * 4 + [
            pltpu.VMEM((_K4C,), jnp.float32),
            pltpu.VMEM((512,), jnp.float32),
            pltpu.VMEM((_CHUNK, _DA), jnp.float32),
            pltpu.VMEM_SHARED((_NP, _DA), jnp.float32),
        ] + [pltpu.SemaphoreType.DMA] * 8,
        compiler_params=pltpu.CompilerParams(needs_layout_passes=False,
                                             use_tc_tiling_on_sc=False),
    )
    return fn(xq[0], xq[1], xq[2], xq[3], src1d, dst2d, e1d, pmax)


def _post_body(q0_ref, q1_ref, q2_ref, q3_ref, batch_ref, bias_ref,
               wlin_ref, blin_ref, out_ref):
    den = q0_ref[...][:_N, _DQ:_DQ + 1] + 1e-16
    h = jnp.concatenate(
        [q0_ref[...][:_N, :_DQ], q1_ref[...][:_N, :_DQ],
         q2_ref[...][:_N, :_DQ], q3_ref[...][:_N, :_DQ]], axis=1) / den
    h = h + bias_ref[...]
    oh = (batch_ref[...] ==
          lax.broadcasted_iota(jnp.int32, (_N, _G), 1)).astype(jnp.float32)
    sums = lax.dot_general(oh, h, (((0,), (0,)), ((), ())),
                           preferred_element_type=jnp.float32)
    counts = lax.dot_general(oh, jnp.ones((_N, 1), jnp.float32),
                             (((0,), (0,)), ((), ())),
                             preferred_element_type=jnp.float32)
    pooled = sums / jnp.maximum(counts, 1.0)
    out_ref[...] = lax.dot_general(pooled, wlin_ref[...],
                                   (((1,), (1,)), ((), ())),
                                   preferred_element_type=jnp.float32) + blin_ref[...]


def _finish(accs, batch, bias_conv, W_lin, b_lin):
    return pl.pallas_call(
        _post_body,
        out_shape=jax.ShapeDtypeStruct((_G, _D), jnp.float32),
    )(accs[0], accs[1], accs[2], accs[3], batch.reshape(_N, 1),
      bias_conv.reshape(1, _D), W_lin, b_lin.reshape(1, _D))


def kernel(x, edge_index, edge_attr, batch, W_l, b_l, W_r, b_r, att,
           bias_conv, W_lin, b_lin):
    del edge_attr
    loop = jnp.arange(_N, dtype=jnp.int32)
    pad = jnp.arange(_EP - _EDGES, dtype=jnp.int32)
    src1d = jnp.concatenate([edge_index[0], loop, pad])
    dst1d = jnp.concatenate([edge_index[1], loop, pad])
    dst2d = dst1d.reshape(_NROW, _K4C)

    xl, xr, q0, q1, q2, q3 = _project(x, W_l, b_l, W_r, b_r)
    e1d, pmax = _edge_scores(xl, xr, src1d, dst1d,
                             att.astype(jnp.bfloat16))
    a0, a1, a2, a3 = _aggregate((q0, q1, q2, q3), src1d, dst2d, e1d, pmax)
    return _finish((a0, a1, a2, a3), batch, bias_conv, W_lin, b_lin)
